# baseline (speedup 1.0000x reference)
import functools

import jax
import jax.numpy as jnp
from jax import lax
from jax.experimental import pallas as pl
from jax.experimental.pallas import tpu as pltpu

TOPK = 64
MAX_ALPHA = 3.0
NEG = -1000000000.0


def _p1_body(qv_ref, wq_ref, dec_ref, z_ref, scores_ref, q_scr):
    i = pl.program_id(0)

    @pl.when(i == 0)
    def _():
        q_scr[...] = lax.dot_general(
            qv_ref[...], wq_ref[...],
            dimension_numbers=(((1,), (1,)), ((), ())),
            preferred_element_type=jnp.float32,
        )

    s = jnp.dot(q_scr[...], dec_ref[...], preferred_element_type=jnp.float32)
    colmax = jnp.max(z_ref[...], axis=0)
    scores_ref[...] = s + jnp.where(colmax > 0.0, 0.0, NEG)[None, :]


def _p2_body(scores_ref, ls_ref, bvec_ref, idx_ref):
    R = scores_ref.shape[0]
    s0 = scores_ref[...]
    scale = jnp.minimum(jnp.exp(ls_ref[0]), 10.0)
    flat = (lax.broadcasted_iota(jnp.int32, (R, 128), 0) * 128
            + lax.broadcasted_iota(jnp.int32, (R, 128), 1))
    flat8 = (lax.broadcasted_iota(jnp.int32, (8, 128), 0) * 128
             + lax.broadcasted_iota(jnp.int32, (8, 128), 1))
    neg_inf = jnp.float32(jnp.finfo(jnp.float32).min)

    def body(i, carry):
        s, bvec, idxacc = carry
        m = jnp.max(s)
        idx = jnp.min(jnp.where(s == m, flat, jnp.int32(2**30)))
        boost = 1.0 + (MAX_ALPHA - 1.0) / (1.0 + jnp.exp(-m * scale))
        hit = flat == idx
        bvec = jnp.where(hit, boost, bvec)
        idxacc = jnp.where(flat8 == i, idx, idxacc)
        s = jnp.where(hit, neg_inf, s)
        return s, bvec, idxacc

    _, bvec, idxacc = lax.fori_loop(
        0, TOPK, body,
        (s0, jnp.ones((R, 128), jnp.float32), jnp.zeros((8, 128), jnp.int32)),
    )
    bvec_ref[...] = bvec
    idx_ref[...] = idxacc


def _p3_body(z_ref, bvec_ref, out_ref):
    out_ref[...] = jnp.where(z_ref[...] > 0.0, bvec_ref[...], 1.0)


def kernel(question_vec, z, decoder_weight, W_q, log_scale):
    qv = question_vec.reshape(1, -1).astype(jnp.float32)
    T, L = z.shape
    H = W_q.shape[0]
    TL = 1024
    nblk = L // TL

    scores = pl.pallas_call(
        _p1_body,
        grid=(nblk,),
        in_specs=[
            pl.BlockSpec((1, H), lambda i: (0, 0)),
            pl.BlockSpec((H, H), lambda i: (0, 0)),
            pl.BlockSpec((H, TL), lambda i: (0, i)),
            pl.BlockSpec((T, TL), lambda i: (0, i)),
        ],
        out_specs=pl.BlockSpec((1, TL), lambda i: (0, i)),
        out_shape=jax.ShapeDtypeStruct((1, L), jnp.float32),
        scratch_shapes=[pltpu.VMEM((1, H), jnp.float32)],
    )(qv, W_q, decoder_weight, z)

    R = L // 128
    bvec, idxs = pl.pallas_call(
        _p2_body,
        in_specs=[
            pl.BlockSpec((R, 128), lambda: (0, 0)),
            pl.BlockSpec(memory_space=pltpu.SMEM),
        ],
        out_specs=[
            pl.BlockSpec((R, 128), lambda: (0, 0)),
            pl.BlockSpec((8, 128), lambda: (0, 0)),
        ],
        out_shape=[
            jax.ShapeDtypeStruct((R, 128), jnp.float32),
            jax.ShapeDtypeStruct((8, 128), jnp.int32),
        ],
    )(scores.reshape(R, 128), log_scale)

    out = pl.pallas_call(
        _p3_body,
        grid=(nblk,),
        in_specs=[
            pl.BlockSpec((T, TL), lambda i: (0, i)),
            pl.BlockSpec((1, TL), lambda i: (0, i)),
        ],
        out_specs=pl.BlockSpec((T, TL), lambda i: (0, i)),
        out_shape=jax.ShapeDtypeStruct((T, L), z.dtype),
    )(z, bvec.reshape(1, L))

    return out
